# baseline (speedup 1.0000x reference)
import jax
import jax.numpy as jnp
from jax import lax
from jax.experimental import pallas as pl
from jax.experimental.pallas import tpu as pltpu
from jax.experimental.pallas import tpu_sc as plsc

M, K, H, B = 4096, 3, 2048, 64
NC, NS = 2, 16
NW = NC * NS
MK = M * K
RPW = MK // NW

_SEGS = [
    (0, B, M),
    (B, M, B),
    (M, M + B, 2 * M),
    (M + B, 2 * M, M + B),
    (2 * M, 2 * M + B, -1),
    (2 * M + B, 3 * M, 2 * M + B),
]


def _worker_copies(w):
  lo, hi = RPW * w, RPW * (w + 1)
  out = []
  for ds_, de_, ss_ in _SEGS:
    a, b_ = max(lo, ds_), min(hi, de_)
    if a < b_:
      src = -1 if ss_ == -1 else ss_ + (a - ds_)
      out.append((a, src, b_ - a))
  return out


def _copy_body(pool_in, new_h, tok_in, ntok, pool_out, tok_out, tok_v):
  w = lax.axis_index("s") * NC + lax.axis_index("c")

  for wi in range(NW):
    copies = _worker_copies(wi)
    if all(src == dst for dst, src, _ in copies if src != -1) and len(copies) == 1:
      continue

    @pl.when(w == wi)
    def _(copies=copies):
      for dst, src, n in copies:
        if src == -1:
          pltpu.sync_copy(new_h, pool_out.at[pl.ds(dst, n)])
        else:
          pltpu.sync_copy(pool_in.at[pl.ds(src, n)],
                          pool_out.at[pl.ds(dst, n)])

  special = [wi for wi in range(NW)
             if not (len(_worker_copies(wi)) == 1
                     and _worker_copies(wi)[0][0] == _worker_copies(wi)[0][1])]
  is_special = (w == special[0])
  for wi in special[1:]:
    is_special = jnp.logical_or(is_special, w == wi)

  @pl.when(jnp.logical_not(is_special))
  def _():
    base = w * RPW
    pltpu.sync_copy(pool_in.at[pl.ds(base, RPW)],
                    pool_out.at[pl.ds(base, RPW)])

  @pl.when(w == 1)
  def _():
    for ds_, de_, ss_ in _SEGS:
      n = de_ - ds_
      if ss_ == -1:
        pltpu.sync_copy(ntok, tok_v.at[pl.ds(ds_, n)])
      else:
        pltpu.sync_copy(tok_in.at[pl.ds(ss_, n)], tok_v.at[pl.ds(ds_, n)])
    pltpu.sync_copy(tok_v, tok_out)


_sc_update = pl.kernel(
    _copy_body,
    out_type=(jax.ShapeDtypeStruct((MK, H), jnp.float32),
              jax.ShapeDtypeStruct((MK,), jnp.int32)),
    mesh=plsc.VectorSubcoreMesh(core_axis_name="c", subcore_axis_name="s"),
    scratch_types=[pltpu.VMEM((MK,), jnp.int32)],
    compiler_params=pltpu.CompilerParams(needs_layout_passes=False),
    name="mtp_pool_update_sc",
)


@jax.jit
def kernel(mem_hidden, new_hidden, slot_ids, mem_tokens, new_tokens):
  del slot_ids
  pool_in = mem_hidden.transpose(1, 0, 2).reshape(MK, H)
  tok_in = mem_tokens.transpose(1, 0).reshape(MK)
  pool_out, tok_out = _sc_update(pool_in, new_hidden, tok_in, new_tokens)
  return (pool_out.reshape(K, M, H).transpose(1, 0, 2),
          tok_out.reshape(K, M).transpose(1, 0))


# speedup vs baseline: 33.5678x; 33.5678x over previous
import jax
import jax.numpy as jnp
from jax import lax
from jax.experimental import pallas as pl
from jax.experimental.pallas import tpu as pltpu
from jax.experimental.pallas import tpu_sc as plsc

M, K, H, B = 4096, 3, 2048, 64
NC, NS = 2, 16
NW = NC * NS
BPW = B // NW
MK = M * K


def _patch_body(pool, new_r, gidx, sidxg, sidxn, sid, ntok, tok, tok_out,
                gidx_v, sidxg_v, sidxn_v, gbuf, nbuf, tok_v, sid_v, ntok_v,
                sem):
  w = lax.axis_index("s") * NC + lax.axis_index("c")

  pltpu.sync_copy(gidx.at[w], gidx_v)
  pltpu.sync_copy(sidxg.at[w], sidxg_v)
  pltpu.sync_copy(sidxn.at[w], sidxn_v)
  pltpu.async_copy(pool.at[gidx_v], gbuf, sem).wait()
  pltpu.sync_copy(new_r.at[w], nbuf)
  pltpu.async_copy(gbuf, pool.at[sidxg_v], sem).wait()
  pltpu.async_copy(nbuf, pool.at[sidxn_v], sem).wait()

  @pl.when(w == 0)
  def _():
    pltpu.sync_copy(tok, tok_v)
    pltpu.sync_copy(sid, sid_v)
    pltpu.sync_copy(ntok, ntok_v)
    for v in range(B // 16):
      s = sid_v[pl.ds(16 * v, 16)]
      g1 = plsc.load_gather(tok_v, [s + M])
      g2 = plsc.load_gather(tok_v, [s + 2 * M])
      nt = ntok_v[pl.ds(16 * v, 16)]
      plsc.store_scatter(tok_v, [s], g1)
      plsc.store_scatter(tok_v, [s + M], g2)
      plsc.store_scatter(tok_v, [s + 2 * M], nt)
    pltpu.sync_copy(tok_v, tok_out)


_sc_patch = pl.kernel(
    _patch_body,
    out_type=jax.ShapeDtypeStruct((MK,), jnp.int32),
    mesh=plsc.VectorSubcoreMesh(core_axis_name="c", subcore_axis_name="s"),
    scratch_types=[
        pltpu.VMEM((2 * BPW,), jnp.int32),
        pltpu.VMEM((2 * BPW,), jnp.int32),
        pltpu.VMEM((BPW,), jnp.int32),
        pltpu.VMEM((2 * BPW, H), jnp.float32),
        pltpu.VMEM((BPW, H), jnp.float32),
        pltpu.VMEM((MK,), jnp.int32),
        pltpu.VMEM((B,), jnp.int32),
        pltpu.VMEM((B,), jnp.int32),
        pltpu.SemaphoreType.DMA,
    ],
    compiler_params=pltpu.CompilerParams(needs_layout_passes=False),
    name="mtp_pool_patch_sc",
)


@jax.jit
def kernel(mem_hidden, new_hidden, slot_ids, mem_tokens, new_tokens):
  s = slot_ids.astype(jnp.int32)
  gidx = jnp.stack([s + M, s + 2 * M], axis=1).reshape(NW, 2 * BPW)
  sidxg = jnp.stack([s, s + M], axis=1).reshape(NW, 2 * BPW)
  sidxn = (s + 2 * M).reshape(NW, BPW)

  pool = jax.new_ref(mem_hidden.transpose(1, 0, 2).reshape(MK, H))
  tok_out = _sc_patch(pool, new_hidden.reshape(NW, BPW, H), gidx, sidxg,
                      sidxn, s, new_tokens,
                      mem_tokens.transpose(1, 0).reshape(MK))
  return (jax.freeze(pool).reshape(K, M, H).transpose(1, 0, 2),
          tok_out.reshape(K, M).transpose(1, 0))
